# baseline (speedup 1.0000x reference)
import functools

import jax
import jax.numpy as jnp
import numpy as np
from jax import lax
from jax.experimental import pallas as pl
from jax.experimental.pallas import tpu as pltpu
from jax.experimental.pallas import tpu_sc as plsc

O_N = 10000
O_PAD = 10240
T_E = 320000
NC, NS = 2, 16
NW = NC * NS
K_IDX = 128
CHUNKS = 80
EPT = K_IDX * CHUNKS
T_PAD = EPT * NW
ROWS_W = CHUNKS
RPT = O_PAD // NS
F32 = jnp.float32
_HI = lax.Precision.HIGHEST

_mesh = plsc.VectorSubcoreMesh(core_axis_name="c", subcore_axis_name="s")

_NOISE = np.asarray(
    jax.random.normal(jax.random.key(42), (10000, 2), jnp.float32))


def _prep_body(obj_ref, w_ref, ps_ref, po_ref):
    obj = obj_ref[...]
    w = w_ref[...]
    ps_ref[...] = jnp.dot(obj, w[:130, :])
    po_ref[...] = jnp.dot(obj, w[130:, :])


def _prep(obj_pad, w1a):
    return pl.pallas_call(
        _prep_body,
        grid=(O_PAD // 2048,),
        in_specs=[
            pl.BlockSpec((2048, 130), lambda i: (i, 0)),
            pl.BlockSpec((260, 16), lambda i: (0, 0)),
        ],
        out_specs=[pl.BlockSpec((2048, 16), lambda i: (i, 0))] * 2,
        out_shape=[jax.ShapeDtypeStruct((O_PAD, 16), F32)] * 2,
    )(obj_pad, w1a)


_GDN = lax.GatherDimensionNumbers(offset_dims=(), collapsed_slice_dims=(0,),
                                  start_index_map=(0,))


def _bcast_lane(vec, lane):
    idx = jnp.full((16, 1), lane, jnp.int32)
    return lax.gather(vec, idx, _GDN, (1,),
                      mode=lax.GatherScatterMode.PROMISE_IN_BOUNDS)


@functools.partial(
    pl.kernel,
    out_type=(
        jax.ShapeDtypeStruct((NC, O_PAD, 16), F32),
        jax.ShapeDtypeStruct((NW, O_PAD), F32),
    ),
    mesh=_mesh,
    compiler_params=pltpu.CompilerParams(use_tc_tiling_on_sc=False,
                                         needs_layout_passes=False),
    scratch_types=[
        pltpu.VMEM((ROWS_W, K_IDX), jnp.int32),
        pltpu.VMEM((ROWS_W, K_IDX), jnp.int32),
        pltpu.VMEM((ROWS_W, K_IDX), F32),
        pltpu.VMEM((40, 16), F32),
        pltpu.VMEM((2, K_IDX, 16), F32),
        pltpu.VMEM((2, K_IDX, 16), F32),
        pltpu.VMEM((2, K_IDX, 16), F32),
        pltpu.VMEM((2, K_IDX, 16), F32),
        pltpu.VMEM((16,), F32),
        pltpu.VMEM((16,), F32),
        pltpu.VMEM((O_PAD,), F32),
        pltpu.VMEM_SHARED((O_PAD, 16), F32),
        pltpu.SemaphoreType.DMA,
        pltpu.SemaphoreType.DMA,
    ],
)
def _edge_fused(ps_hbm, po_hbm, eidx_hbm, ew_hbm, wconst_hbm,
                zeros_hbm, pp_hbm, cnt_hbm,
                sidx_v, oidx_v, ew_v, wconst_v, psb, pob, outs, outo,
                h1b0, h1b1, cnt_v, accp, gsem, ssem):
    cid = lax.axis_index("c")
    sid = lax.axis_index("s")
    wid = sid * NC + cid
    rowbase = wid * ROWS_W
    rbase = sid * RPT

    pltpu.sync_copy(eidx_hbm.at[0, pl.ds(rowbase, ROWS_W)], sidx_v)
    pltpu.sync_copy(eidx_hbm.at[1, pl.ds(rowbase, ROWS_W)], oidx_v)
    pltpu.sync_copy(ew_hbm.at[pl.ds(rowbase, ROWS_W)], ew_v)
    pltpu.sync_copy(wconst_hbm, wconst_v)

    def _zc(i, carry):
        cnt_v[pl.ds(i * 16, 16)] = jnp.zeros((16,), F32)
        return carry

    lax.fori_loop(0, O_PAD // 16, _zc, 0)
    w1bs = [wconst_v[k] for k in range(16)]
    w1bo = [wconst_v[16 + k] for k in range(16)]
    b1a_v = wconst_v[32]
    b1bs_v = wconst_v[33]
    b1bo_v = wconst_v[34]

    def compute_chunk(c, slot):
        def group(g, carry):
            ewg = ew_v[c, pl.ds(g * 16, 16)]
            for l in range(16):
                e = g * 16 + l
                h1b = h1b0 if l % 2 == 0 else h1b1
                w = _bcast_lane(ewg, l)
                h1 = jnp.maximum((psb[slot, e] + pob[slot, e]) * w + b1a_v,
                                 0.0)
                h1b[...] = h1
                accs = b1bs_v
                acco = b1bo_v
                for k in range(8):
                    hk = _bcast_lane(h1, k)
                    accs = accs + hk * w1bs[k]
                    acco = acco + hk * w1bo[k]
                for k in range(8, 16):
                    hk = plsc.load_gather(h1b,
                                          [jnp.full((16,), k, jnp.int32)])
                    accs = accs + hk * w1bs[k]
                    acco = acco + hk * w1bo[k]
                outs[slot, e] = jnp.maximum(accs, 0.0)
                outo[slot, e] = jnp.maximum(acco, 0.0)
            ones16 = jnp.ones((16,), F32)
            plsc.addupdate_scatter(cnt_v, [sidx_v[c, pl.ds(g * 16, 16)]],
                                   ones16)
            plsc.addupdate_scatter(cnt_v, [oidx_v[c, pl.ds(g * 16, 16)]],
                                   ones16)
            return carry

        lax.fori_loop(0, 8, group, 0)

    def fire_gather(c, slot):
        pltpu.async_copy(ps_hbm.at[sidx_v.at[c]], psb.at[slot], gsem)
        pltpu.async_copy(po_hbm.at[oidx_v.at[c]], pob.at[slot], gsem)

    def wait_gather(c, slot):
        pltpu.make_async_copy(ps_hbm.at[sidx_v.at[c]], psb.at[slot],
                              gsem).wait()
        pltpu.make_async_copy(po_hbm.at[oidx_v.at[c]], pob.at[slot],
                              gsem).wait()

    def fire_scatter(c, slot, accp):
        pltpu.async_copy(outs.at[slot], accp.at[sidx_v.at[c]], ssem, add=True)
        pltpu.async_copy(outo.at[slot], accp.at[oidx_v.at[c]], ssem, add=True)

    def wait_scatter(c, slot, accp):
        pltpu.make_async_copy(outs.at[slot], accp.at[sidx_v.at[c]],
                              ssem).wait()
        pltpu.make_async_copy(outo.at[slot], accp.at[oidx_v.at[c]],
                              ssem).wait()

    def run():
        pltpu.sync_copy(zeros_hbm, accp.at[pl.ds(rbase, RPT)])
        plsc.subcore_barrier()

        fire_gather(0, 0)
        fire_gather(1, 1)

        def body(i, carry):
            for sub in range(2):
                c = i * 2 + sub
                wait_gather(c, sub)
                compute_chunk(c, sub)

                @pl.when(c >= 2)
                def _():
                    wait_scatter(c - 2, sub, accp)

                fire_scatter(c, sub, accp)

                @pl.when(c + 2 < CHUNKS)
                def _():
                    fire_gather(c + 2, sub)
            return carry

        lax.fori_loop(0, CHUNKS // 2, body, 0)
        wait_scatter(CHUNKS - 2, 0, accp)
        wait_scatter(CHUNKS - 1, 1, accp)
        plsc.subcore_barrier()
        pltpu.sync_copy(accp.at[pl.ds(rbase, RPT)],
                        pp_hbm.at[cid, pl.ds(rbase, RPT)])
        pltpu.sync_copy(cnt_v, cnt_hbm.at[wid])

    run()


def _final_body(pp_ref, pc_ref, w2a_ref, b2a_ref, w2b_ref, b2b_ref, out_ref):
    pool = pp_ref[0] + pp_ref[1]
    cnt = jnp.sum(pc_ref[...], axis=0)[:, None]
    pooled = pool / jnp.maximum(cnt, 1.0)
    h2 = jnp.maximum(jnp.dot(pooled, w2a_ref[...]) + b2a_ref[...], 0.0)
    out_ref[...] = jnp.maximum(jnp.dot(h2, w2b_ref[...]) + b2b_ref[...], 0.0)


def _final(pp, pc, w2a, b2a, w2b, b2b):
    return pl.pallas_call(
        _final_body,
        grid=(O_PAD // 2048,),
        in_specs=[
            pl.BlockSpec((NC, 2048, 16), lambda i: (0, i, 0)),
            pl.BlockSpec((NW, 2048), lambda i: (0, i)),
            pl.BlockSpec((16, 16), lambda i: (0, 0)),
            pl.BlockSpec((1, 16), lambda i: (0, 0)),
            pl.BlockSpec((16, 130), lambda i: (0, 0)),
            pl.BlockSpec((1, 130), lambda i: (0, 0)),
        ],
        out_specs=pl.BlockSpec((2048, 130), lambda i: (i, 0)),
        out_shape=jax.ShapeDtypeStruct((O_N, 130), F32),
    )(pp, pc, w2a, b2a, w2b, b2b)


def kernel(object_embeddings, edge_weights, edges, W1a, b1a, W1b, b1b,
           W2a, b2a, W2b, b2b):
    dtype = object_embeddings.dtype
    noise = jnp.asarray(_NOISE, dtype)
    obj = jnp.concatenate([object_embeddings, noise], axis=1)
    obj_pad = jnp.pad(obj, ((0, O_PAD - O_N), (0, 0)))

    eidx = jnp.pad(edges.astype(jnp.int32).T, ((0, 0), (0, T_PAD - T_E)),
                   constant_values=O_N).reshape(2, T_PAD // 128, 128)
    ew2 = jnp.pad(edge_weights.reshape(T_E), (0, T_PAD - T_E)).reshape(
        T_PAD // 128, 128)

    ps, po = _prep(obj_pad, W1a)

    wconst = jnp.concatenate([
        W1b[:, :16],
        W1b[:, 16:],
        b1a.reshape(1, 16),
        b1b[:16].reshape(1, 16),
        b1b[16:].reshape(1, 16),
        jnp.zeros((5, 16), F32),
    ], axis=0)
    zeros_rows = jnp.zeros((RPT, 16), F32)

    pp, pc = _edge_fused(ps, po, eidx, ew2, wconst, zeros_rows)

    return _final(pp, pc, W2a, b2a.reshape(1, 16), W2b,
                  b2b.reshape(1, 130))


# speedup vs baseline: 1.0636x; 1.0636x over previous
import functools

import jax
import jax.numpy as jnp
import numpy as np
from jax import lax
from jax.experimental import pallas as pl
from jax.experimental.pallas import tpu as pltpu
from jax.experimental.pallas import tpu_sc as plsc

O_N = 10000
O_PAD = 10240
T_E = 320000
NC, NS = 2, 16
NW = NC * NS
K_IDX = 128
CHUNKS = 80
EPT = K_IDX * CHUNKS
T_PAD = EPT * NW
ROWS_W = CHUNKS
RPT = O_PAD // NS
F32 = jnp.float32
_HI = lax.Precision.HIGHEST

_mesh = plsc.VectorSubcoreMesh(core_axis_name="c", subcore_axis_name="s")

_NOISE = np.asarray(
    jax.random.normal(jax.random.key(42), (10000, 2), jnp.float32))


def _prep_body(obj_ref, w_ref, ps_ref, po_ref):
    obj = obj_ref[...]
    w = w_ref[...]
    ps_ref[...] = jnp.dot(obj, w[:130, :])
    po_ref[...] = jnp.dot(obj, w[130:, :])


def _prep(obj_pad, w1a):
    return pl.pallas_call(
        _prep_body,
        grid=(O_PAD // 2048,),
        in_specs=[
            pl.BlockSpec((2048, 130), lambda i: (i, 0)),
            pl.BlockSpec((260, 16), lambda i: (0, 0)),
        ],
        out_specs=[pl.BlockSpec((2048, 16), lambda i: (i, 0))] * 2,
        out_shape=[jax.ShapeDtypeStruct((O_PAD, 16), F32)] * 2,
    )(obj_pad, w1a)


_GDN = lax.GatherDimensionNumbers(offset_dims=(), collapsed_slice_dims=(0,),
                                  start_index_map=(0,))


def _bcast_lane(vec, lane):
    idx = jnp.full((16, 1), lane, jnp.int32)
    return lax.gather(vec, idx, _GDN, (1,),
                      mode=lax.GatherScatterMode.PROMISE_IN_BOUNDS)


@functools.partial(
    pl.kernel,
    out_type=(
        jax.ShapeDtypeStruct((NC, O_PAD, 16), F32),
        jax.ShapeDtypeStruct((NC, O_PAD, 16), F32),
    ),
    mesh=_mesh,
    compiler_params=pltpu.CompilerParams(use_tc_tiling_on_sc=False,
                                         needs_layout_passes=False),
    scratch_types=[
        pltpu.VMEM((ROWS_W, K_IDX), jnp.int32),
        pltpu.VMEM((ROWS_W, K_IDX), jnp.int32),
        pltpu.VMEM((ROWS_W, K_IDX), F32),
        pltpu.VMEM((40, 16), F32),
        pltpu.VMEM((K_IDX, 16), F32),
        pltpu.VMEM((2, K_IDX, 16), F32),
        pltpu.VMEM((2, K_IDX, 16), F32),
        pltpu.VMEM((2, K_IDX, 16), F32),
        pltpu.VMEM((2, K_IDX, 16), F32),
        pltpu.VMEM((16,), F32),
        pltpu.VMEM((16,), F32),
        pltpu.VMEM_SHARED((O_PAD, 16), F32),
        pltpu.VMEM_SHARED((O_PAD, 16), F32),
        pltpu.SemaphoreType.DMA,
        pltpu.SemaphoreType.DMA,
    ],
)
def _edge_fused(ps_hbm, po_hbm, eidx_hbm, ew_hbm, wconst_hbm,
                ones_hbm, zeros_hbm, pp_hbm, pc_hbm,
                sidx_v, oidx_v, ew_v, wconst_v, ones_v, psb, pob, outs, outo,
                h1b0, h1b1, accp, accc, gsem, ssem):
    cid = lax.axis_index("c")
    sid = lax.axis_index("s")
    wid = sid * NC + cid
    rowbase = wid * ROWS_W
    rbase = sid * RPT

    pltpu.sync_copy(eidx_hbm.at[0, pl.ds(rowbase, ROWS_W)], sidx_v)
    pltpu.sync_copy(eidx_hbm.at[1, pl.ds(rowbase, ROWS_W)], oidx_v)
    pltpu.sync_copy(ew_hbm.at[pl.ds(rowbase, ROWS_W)], ew_v)
    pltpu.sync_copy(wconst_hbm, wconst_v)
    pltpu.sync_copy(ones_hbm, ones_v)
    w1bs = [wconst_v[k] for k in range(16)]
    w1bo = [wconst_v[16 + k] for k in range(16)]
    b1a_v = wconst_v[32]
    b1bs_v = wconst_v[33]
    b1bo_v = wconst_v[34]

    def compute_chunk(c, slot):
        def group(g, carry):
            ewg = ew_v[c, pl.ds(g * 16, 16)]
            for l in range(16):
                e = g * 16 + l
                h1b = h1b0 if l % 2 == 0 else h1b1
                w = _bcast_lane(ewg, l)
                h1 = jnp.maximum((psb[slot, e] + pob[slot, e]) * w + b1a_v,
                                 0.0)
                h1b[...] = h1
                accs = b1bs_v
                acco = b1bo_v
                for k in range(8):
                    hk = _bcast_lane(h1, k)
                    accs = accs + hk * w1bs[k]
                    acco = acco + hk * w1bo[k]
                for k in range(8, 16):
                    hk = plsc.load_gather(h1b,
                                          [jnp.full((16,), k, jnp.int32)])
                    accs = accs + hk * w1bs[k]
                    acco = acco + hk * w1bo[k]
                outs[slot, e] = jnp.maximum(accs, 0.0)
                outo[slot, e] = jnp.maximum(acco, 0.0)
            return carry

        lax.fori_loop(0, 8, group, 0)

    def fire_gather(c, slot):
        pltpu.async_copy(ps_hbm.at[sidx_v.at[c]], psb.at[slot], gsem)
        pltpu.async_copy(po_hbm.at[oidx_v.at[c]], pob.at[slot], gsem)

    def wait_gather(c, slot):
        pltpu.make_async_copy(ps_hbm.at[sidx_v.at[c]], psb.at[slot],
                              gsem).wait()
        pltpu.make_async_copy(po_hbm.at[oidx_v.at[c]], pob.at[slot],
                              gsem).wait()

    def fire_scatter(c, slot, accp, accc):
        pltpu.async_copy(outs.at[slot], accp.at[sidx_v.at[c]], ssem, add=True)
        pltpu.async_copy(outo.at[slot], accp.at[oidx_v.at[c]], ssem, add=True)
        pltpu.async_copy(ones_v, accc.at[sidx_v.at[c]], ssem, add=True)
        pltpu.async_copy(ones_v, accc.at[oidx_v.at[c]], ssem, add=True)

    def wait_scatter(c, slot, accp, accc):
        pltpu.make_async_copy(outs.at[slot], accp.at[sidx_v.at[c]],
                              ssem).wait()
        pltpu.make_async_copy(outo.at[slot], accp.at[oidx_v.at[c]],
                              ssem).wait()
        pltpu.make_async_copy(ones_v, accc.at[sidx_v.at[c]], ssem).wait()
        pltpu.make_async_copy(ones_v, accc.at[oidx_v.at[c]], ssem).wait()

    def run():
        pltpu.sync_copy(zeros_hbm, accp.at[pl.ds(rbase, RPT)])
        pltpu.sync_copy(zeros_hbm, accc.at[pl.ds(rbase, RPT)])
        plsc.subcore_barrier()

        fire_gather(0, 0)
        fire_gather(1, 1)

        def body(i, carry):
            for sub in range(2):
                c = i * 2 + sub
                wait_gather(c, sub)
                compute_chunk(c, sub)

                @pl.when(c >= 2)
                def _():
                    wait_scatter(c - 2, sub, accp, accc)

                fire_scatter(c, sub, accp, accc)

                @pl.when(c + 2 < CHUNKS)
                def _():
                    fire_gather(c + 2, sub)
            return carry

        lax.fori_loop(0, CHUNKS // 2, body, 0)
        wait_scatter(CHUNKS - 2, 0, accp, accc)
        wait_scatter(CHUNKS - 1, 1, accp, accc)
        plsc.subcore_barrier()
        pltpu.sync_copy(accp.at[pl.ds(rbase, RPT)],
                        pp_hbm.at[cid, pl.ds(rbase, RPT)])
        pltpu.sync_copy(accc.at[pl.ds(rbase, RPT)],
                        pc_hbm.at[cid, pl.ds(rbase, RPT)])

    run()


def _final_body(pp_ref, pc_ref, w2a_ref, b2a_ref, w2b_ref, b2b_ref, out_ref):
    pool = pp_ref[0] + pp_ref[1]
    cnt = pc_ref[0, :, 0:1] + pc_ref[1, :, 0:1]
    pooled = pool / jnp.maximum(cnt, 1.0)
    h2 = jnp.maximum(jnp.dot(pooled, w2a_ref[...]) + b2a_ref[...], 0.0)
    out_ref[...] = jnp.maximum(jnp.dot(h2, w2b_ref[...]) + b2b_ref[...], 0.0)


def _final(pp, pc, w2a, b2a, w2b, b2b):
    return pl.pallas_call(
        _final_body,
        grid=(O_PAD // 2048,),
        in_specs=[
            pl.BlockSpec((NC, 2048, 16), lambda i: (0, i, 0)),
            pl.BlockSpec((NC, 2048, 16), lambda i: (0, i, 0)),
            pl.BlockSpec((16, 16), lambda i: (0, 0)),
            pl.BlockSpec((1, 16), lambda i: (0, 0)),
            pl.BlockSpec((16, 130), lambda i: (0, 0)),
            pl.BlockSpec((1, 130), lambda i: (0, 0)),
        ],
        out_specs=pl.BlockSpec((2048, 130), lambda i: (i, 0)),
        out_shape=jax.ShapeDtypeStruct((O_N, 130), F32),
    )(pp, pc, w2a, b2a, w2b, b2b)


def kernel(object_embeddings, edge_weights, edges, W1a, b1a, W1b, b1b,
           W2a, b2a, W2b, b2b):
    dtype = object_embeddings.dtype
    noise = jnp.asarray(_NOISE, dtype)
    obj = jnp.concatenate([object_embeddings, noise], axis=1)
    obj_pad = jnp.pad(obj, ((0, O_PAD - O_N), (0, 0)))

    eidx = jnp.pad(edges.astype(jnp.int32).T, ((0, 0), (0, T_PAD - T_E)),
                   constant_values=O_N).reshape(2, T_PAD // 128, 128)
    ew2 = jnp.pad(edge_weights.reshape(T_E), (0, T_PAD - T_E)).reshape(
        T_PAD // 128, 128)

    ps, po = _prep(obj_pad, W1a)

    wconst = jnp.concatenate([
        W1b[:, :16],
        W1b[:, 16:],
        b1a.reshape(1, 16),
        b1b[:16].reshape(1, 16),
        b1b[16:].reshape(1, 16),
        jnp.zeros((5, 16), F32),
    ], axis=0)
    ones_rows = jnp.zeros((K_IDX, 16), F32).at[:, 0].set(1.0)
    zeros_rows = jnp.zeros((RPT, 16), F32)

    pp, pc = _edge_fused(ps, po, eidx, ew2, wconst, ones_rows, zeros_rows)

    return _final(pp, pc, W2a, b2a.reshape(1, 16), W2b,
                  b2b.reshape(1, 130))
